# baseline (speedup 1.0000x reference)
import jax
import jax.numpy as jnp
from jax.experimental import pallas as pl
from jax.experimental.pallas import tpu as pltpu

_EPS = 1e-5


def _decimate_kernel(x, sel):
    N, Cin, H, W = x.shape
    Ho, Wo = H // 2, W // 2
    _NCHUNK = 7 if H % (7 * 16) == 0 else 1
    hc = H // _NCHUNK
    hco = hc // 2

    def body(*refs):
        x_refs = refs[:_NCHUNK]
        s_ref = refs[_NCHUNK]
        o_ref = refs[_NCHUNK + 1]
        for k in range(_NCHUNK):
            xh = x_refs[k][0, :, ::2, :]
            z = jnp.maximum(xh, 0.0)
            z2 = z.reshape(Cin * hco, W)
            zd2 = jnp.dot(z2, s_ref[...],
                          preferred_element_type=jnp.float32)
            o_ref[0, :, k * hco:(k + 1) * hco, :] = zd2.reshape(Cin, hco, Wo)

    return pl.pallas_call(
        body,
        out_shape=jax.ShapeDtypeStruct((N, Cin, Ho, Wo), x.dtype),
        grid=(N,),
        in_specs=[
            pl.BlockSpec((1, Cin, hc, W),
                         lambda i, k=k: (i, 0, k, 0))
            for k in range(_NCHUNK)
        ] + [pl.BlockSpec((W, Wo), lambda i: (0, 0))],
        out_specs=pl.BlockSpec((1, Cin, Ho, Wo), lambda i: (i, 0, 0, 0)),
        compiler_params=pltpu.CompilerParams(
            dimension_semantics=("parallel",),
            vmem_limit_bytes=48 * 1024 * 1024,
        ),
    )(*([x] * _NCHUNK + [sel]))


def _conv_bn_kernel(xs, w_mat, g, b, *, N, Cout, M, total):
    Cin = xs.shape[1]

    def body(x_ref, w_ref, g_ref, b_ref, o_ref, y_ref, s_ref, q_ref,
             sc_ref, sh_ref):
        step = pl.program_id(0)

        @pl.when(step == 0)
        def _init():
            s_ref[...] = jnp.zeros_like(s_ref)
            q_ref[...] = jnp.zeros_like(q_ref)

        @pl.when(step < N)
        def _compute():
            xn = x_ref[0]
            yn = jnp.dot(w_ref[...], xn, preferred_element_type=jnp.float32)
            y_ref[pl.ds(step, 1)] = yn[None]
            s_ref[...] += jnp.sum(yn, axis=1, keepdims=True)
            q_ref[...] += jnp.sum(yn * yn, axis=1, keepdims=True)

        @pl.when(step == N)
        def _finalize():
            inv_cnt = 1.0 / float(total)
            mean = s_ref[...] * inv_cnt
            var = jnp.maximum(q_ref[...] * inv_cnt - mean * mean, 0.0)
            sc = g_ref[...] * jax.lax.rsqrt(var + _EPS)
            sc_ref[...] = sc
            sh_ref[...] = b_ref[...] - mean * sc

        @pl.when(step >= N)
        def _write():
            n = step - N
            yn = y_ref[pl.ds(n, 1)]
            o_ref[...] = (yn * sc_ref[...][None] + sh_ref[...][None]
                          ).astype(o_ref.dtype)

    return pl.pallas_call(
        body,
        out_shape=jax.ShapeDtypeStruct((N, Cout, M), xs.dtype),
        grid=(2 * N,),
        in_specs=[
            pl.BlockSpec((1, Cin, M),
                         lambda i: (jnp.where(i < N, i, N - 1), 0, 0)),
            pl.BlockSpec((Cout, Cin), lambda i: (0, 0)),
            pl.BlockSpec((Cout, 1), lambda i: (0, 0)),
            pl.BlockSpec((Cout, 1), lambda i: (0, 0)),
        ],
        out_specs=pl.BlockSpec(
            (1, Cout, M), lambda i: (jnp.where(i < N, 0, i - N), 0, 0)),
        scratch_shapes=[
            pltpu.VMEM((N, Cout, M), jnp.float32),
            pltpu.VMEM((Cout, 1), jnp.float32),
            pltpu.VMEM((Cout, 1), jnp.float32),
            pltpu.VMEM((Cout, 1), jnp.float32),
            pltpu.VMEM((Cout, 1), jnp.float32),
        ],
        compiler_params=pltpu.CompilerParams(
            dimension_semantics=("arbitrary",),
            vmem_limit_bytes=52 * 1024 * 1024,
        ),
    )(xs, w_mat, g, b)


def kernel(x_nchw, w_oihw, gamma, beta):
    N, Cin, H, W = x_nchw.shape
    Cout = w_oihw.shape[0]
    Ho, Wo = (H + 1) // 2, (W + 1) // 2
    M = Ho * Wo
    total = N * M

    w_mat = w_oihw.reshape(Cout, Cin).astype(jnp.float32)
    g = gamma.reshape(Cout, 1).astype(jnp.float32)
    b = beta.reshape(Cout, 1).astype(jnp.float32)
    sel = (jax.lax.broadcasted_iota(jnp.int32, (W, Wo), 0)
           == 2 * jax.lax.broadcasted_iota(jnp.int32, (W, Wo), 1)
           ).astype(jnp.float32)

    xs = _decimate_kernel(x_nchw, sel).reshape(N, Cin, M)
    out_flat = _conv_bn_kernel(xs, w_mat, g, b, N=N, Cout=Cout, M=M,
                               total=total)
    return out_flat.reshape(N, Cout, Ho, Wo)


# speedup vs baseline: 1.0435x; 1.0435x over previous
import jax
import jax.numpy as jnp
from jax.experimental import pallas as pl
from jax.experimental.pallas import tpu as pltpu

_EPS = 1e-5


def _decimate_kernel(x, sel):
    N, Cin, Ho, _, W = x.shape
    Wo = W // 2

    def body(x_hbm, s_ref, o_ref, buf, sem):
        i = pl.program_id(0)

        def _start(img, slot):
            pltpu.make_async_copy(
                x_hbm.at[img, :, :, 0, :], buf.at[slot], sem.at[slot],
            ).start()

        @pl.when(i == 0)
        def _():
            _start(0, 0)

        @pl.when(i + 1 < N)
        def _():
            _start(i + 1, (i + 1) % 2)

        slot = i % 2
        pltpu.make_async_copy(buf.at[slot], buf.at[slot], sem.at[slot]).wait()

        z = jnp.maximum(buf[slot], 0.0)
        z2 = z.reshape(Cin * Ho, W)
        zd2 = jnp.dot(z2, s_ref[...],
                      preferred_element_type=jnp.float32)
        o_ref[0] = zd2.reshape(Cin, Ho, Wo)

    return pl.pallas_call(
        body,
        out_shape=jax.ShapeDtypeStruct((N, Cin, Ho, Wo), x.dtype),
        grid=(N,),
        in_specs=[
            pl.BlockSpec(memory_space=pltpu.MemorySpace.HBM),
            pl.BlockSpec((W, Wo), lambda i: (0, 0)),
        ],
        out_specs=pl.BlockSpec((1, Cin, Ho, Wo), lambda i: (i, 0, 0, 0)),
        scratch_shapes=[
            pltpu.VMEM((2, Cin, Ho, W), jnp.float32),
            pltpu.SemaphoreType.DMA((2,)),
        ],
        compiler_params=pltpu.CompilerParams(
            dimension_semantics=("arbitrary",),
            vmem_limit_bytes=48 * 1024 * 1024,
        ),
    )(x, sel)


def _conv_bn_kernel(xs, w_mat, g, b, *, N, Cout, M, total):
    Cin = xs.shape[1]

    def body(x_ref, w_ref, g_ref, b_ref, o_ref, y_ref, s_ref, q_ref,
             sc_ref, sh_ref):
        step = pl.program_id(0)

        @pl.when(step == 0)
        def _init():
            s_ref[...] = jnp.zeros_like(s_ref)
            q_ref[...] = jnp.zeros_like(q_ref)

        @pl.when(step < N)
        def _compute():
            xn = x_ref[0]
            yn = jnp.dot(w_ref[...], xn, preferred_element_type=jnp.float32)
            y_ref[pl.ds(step, 1)] = yn[None]
            s_ref[...] += jnp.sum(yn, axis=1, keepdims=True)
            q_ref[...] += jnp.sum(yn * yn, axis=1, keepdims=True)

        @pl.when(step == N)
        def _finalize():
            inv_cnt = 1.0 / float(total)
            mean = s_ref[...] * inv_cnt
            var = jnp.maximum(q_ref[...] * inv_cnt - mean * mean, 0.0)
            sc = g_ref[...] * jax.lax.rsqrt(var + _EPS)
            sc_ref[...] = sc
            sh_ref[...] = b_ref[...] - mean * sc

        @pl.when(step >= N)
        def _write():
            n = step - N
            yn = y_ref[pl.ds(n, 1)]
            o_ref[...] = (yn * sc_ref[...][None] + sh_ref[...][None]
                          ).astype(o_ref.dtype)

    return pl.pallas_call(
        body,
        out_shape=jax.ShapeDtypeStruct((N, Cout, M), xs.dtype),
        grid=(2 * N,),
        in_specs=[
            pl.BlockSpec((1, Cin, M),
                         lambda i: (jnp.where(i < N, i, N - 1), 0, 0)),
            pl.BlockSpec((Cout, Cin), lambda i: (0, 0)),
            pl.BlockSpec((Cout, 1), lambda i: (0, 0)),
            pl.BlockSpec((Cout, 1), lambda i: (0, 0)),
        ],
        out_specs=pl.BlockSpec(
            (1, Cout, M), lambda i: (jnp.where(i < N, 0, i - N), 0, 0)),
        scratch_shapes=[
            pltpu.VMEM((N, Cout, M), jnp.float32),
            pltpu.VMEM((Cout, 1), jnp.float32),
            pltpu.VMEM((Cout, 1), jnp.float32),
            pltpu.VMEM((Cout, 1), jnp.float32),
            pltpu.VMEM((Cout, 1), jnp.float32),
        ],
        compiler_params=pltpu.CompilerParams(
            dimension_semantics=("arbitrary",),
            vmem_limit_bytes=52 * 1024 * 1024,
        ),
    )(xs, w_mat, g, b)


def kernel(x_nchw, w_oihw, gamma, beta):
    N, Cin, H, W = x_nchw.shape
    Cout = w_oihw.shape[0]
    Ho, Wo = (H + 1) // 2, (W + 1) // 2
    M = Ho * Wo
    total = N * M

    w_mat = w_oihw.reshape(Cout, Cin).astype(jnp.float32)
    g = gamma.reshape(Cout, 1).astype(jnp.float32)
    b = beta.reshape(Cout, 1).astype(jnp.float32)
    sel = (jax.lax.broadcasted_iota(jnp.int32, (W, Wo), 0)
           == 2 * jax.lax.broadcasted_iota(jnp.int32, (W, Wo), 1)
           ).astype(jnp.float32)

    xv = x_nchw.reshape(N, Cin, Ho, 2, W)
    xs = _decimate_kernel(xv, sel).reshape(N, Cin, M)
    out_flat = _conv_bn_kernel(xs, w_mat, g, b, N=N, Cout=Cout, M=M,
                               total=total)
    return out_flat.reshape(N, Cout, Ho, Wo)


# speedup vs baseline: 1.0959x; 1.0502x over previous
import jax
import jax.numpy as jnp
from jax.experimental import pallas as pl
from jax.experimental.pallas import tpu as pltpu

_EPS = 1e-5


def _decimate_kernel(x, sel):
    N, Cin, Ho, _, W = x.shape
    Wo = W // 2

    def body(x_hbm, s_ref, o_ref, buf, sem):
        i = pl.program_id(0)

        def _start(img, slot):
            pltpu.make_async_copy(
                x_hbm.at[img, :, :, 0, :], buf.at[slot], sem.at[slot],
            ).start()

        @pl.when(i == 0)
        def _():
            for img in range(N):
                _start(img, img)

        slot = i
        pltpu.make_async_copy(buf.at[slot], buf.at[slot], sem.at[slot]).wait()

        z = jnp.maximum(buf[slot], 0.0)
        z2 = z.reshape(Cin * Ho, W)
        zd2 = jnp.dot(z2, s_ref[...],
                      preferred_element_type=jnp.float32)
        o_ref[0] = zd2.reshape(Cin, Ho, Wo)

    return pl.pallas_call(
        body,
        out_shape=jax.ShapeDtypeStruct((N, Cin, Ho, Wo), x.dtype),
        grid=(N,),
        in_specs=[
            pl.BlockSpec(memory_space=pltpu.MemorySpace.HBM),
            pl.BlockSpec((W, Wo), lambda i: (0, 0)),
        ],
        out_specs=pl.BlockSpec((1, Cin, Ho, Wo), lambda i: (i, 0, 0, 0)),
        scratch_shapes=[
            pltpu.VMEM((N, Cin, Ho, W), jnp.float32),
            pltpu.SemaphoreType.DMA((N,)),
        ],
        compiler_params=pltpu.CompilerParams(
            dimension_semantics=("arbitrary",),
            vmem_limit_bytes=48 * 1024 * 1024,
        ),
    )(x, sel)


def _conv_bn_kernel(xs, w_mat, g, b, *, N, Cout, M, total):
    Cin = xs.shape[1]

    def body(x_ref, w_ref, g_ref, b_ref, o_ref, y_ref, s_ref, q_ref,
             sc_ref, sh_ref):
        step = pl.program_id(0)

        @pl.when(step == 0)
        def _init():
            s_ref[...] = jnp.zeros_like(s_ref)
            q_ref[...] = jnp.zeros_like(q_ref)

        @pl.when(step < N)
        def _compute():
            xn = x_ref[0]
            yn = jnp.dot(w_ref[...], xn, preferred_element_type=jnp.float32)
            y_ref[pl.ds(step, 1)] = yn[None]
            s_ref[...] += jnp.sum(yn, axis=1, keepdims=True)
            q_ref[...] += jnp.sum(yn * yn, axis=1, keepdims=True)

        @pl.when(step == N)
        def _finalize():
            inv_cnt = 1.0 / float(total)
            mean = s_ref[...] * inv_cnt
            var = jnp.maximum(q_ref[...] * inv_cnt - mean * mean, 0.0)
            sc = g_ref[...] * jax.lax.rsqrt(var + _EPS)
            sc_ref[...] = sc
            sh_ref[...] = b_ref[...] - mean * sc

        @pl.when(step >= N)
        def _write():
            n = step - N
            yn = y_ref[pl.ds(n, 1)]
            o_ref[...] = (yn * sc_ref[...][None] + sh_ref[...][None]
                          ).astype(o_ref.dtype)

    return pl.pallas_call(
        body,
        out_shape=jax.ShapeDtypeStruct((N, Cout, M), xs.dtype),
        grid=(2 * N,),
        in_specs=[
            pl.BlockSpec((1, Cin, M),
                         lambda i: (jnp.where(i < N, i, N - 1), 0, 0)),
            pl.BlockSpec((Cout, Cin), lambda i: (0, 0)),
            pl.BlockSpec((Cout, 1), lambda i: (0, 0)),
            pl.BlockSpec((Cout, 1), lambda i: (0, 0)),
        ],
        out_specs=pl.BlockSpec(
            (1, Cout, M), lambda i: (jnp.where(i < N, 0, i - N), 0, 0)),
        scratch_shapes=[
            pltpu.VMEM((N, Cout, M), jnp.float32),
            pltpu.VMEM((Cout, 1), jnp.float32),
            pltpu.VMEM((Cout, 1), jnp.float32),
            pltpu.VMEM((Cout, 1), jnp.float32),
            pltpu.VMEM((Cout, 1), jnp.float32),
        ],
        compiler_params=pltpu.CompilerParams(
            dimension_semantics=("arbitrary",),
            vmem_limit_bytes=52 * 1024 * 1024,
        ),
    )(xs, w_mat, g, b)


def kernel(x_nchw, w_oihw, gamma, beta):
    N, Cin, H, W = x_nchw.shape
    Cout = w_oihw.shape[0]
    Ho, Wo = (H + 1) // 2, (W + 1) // 2
    M = Ho * Wo
    total = N * M

    w_mat = w_oihw.reshape(Cout, Cin).astype(jnp.float32)
    g = gamma.reshape(Cout, 1).astype(jnp.float32)
    b = beta.reshape(Cout, 1).astype(jnp.float32)
    sel = (jax.lax.broadcasted_iota(jnp.int32, (W, Wo), 0)
           == 2 * jax.lax.broadcasted_iota(jnp.int32, (W, Wo), 1)
           ).astype(jnp.float32)

    xv = x_nchw.reshape(N, Cin, Ho, 2, W)
    xs = _decimate_kernel(xv, sel).reshape(N, Cin, M)
    out_flat = _conv_bn_kernel(xs, w_mat, g, b, N=N, Cout=Cout, M=M,
                               total=total)
    return out_flat.reshape(N, Cout, Ho, Wo)
